# baseline (device time: 16051 ns/iter reference)
import jax
import jax.numpy as jnp
from jax import lax
from jax.experimental import pallas as pl
from jax.experimental.pallas import tpu as pltpu

N_DEV = 4
B, SQ, D = 2, 128, 512
H_LOC = 8
DH = 64
SCALE = 0.125


def kernel(x, Wq, Wo, K_ext, V_ext):
    my = lax.axis_index("i")
    K_loc = jnp.transpose(
        lax.dynamic_slice_in_dim(K_ext, my * H_LOC, H_LOC, axis=2), (0, 2, 1, 3)
    ).astype(jnp.bfloat16)
    V_loc = jnp.transpose(
        lax.dynamic_slice_in_dim(V_ext, my * H_LOC, H_LOC, axis=2), (0, 2, 1, 3)
    ).astype(jnp.bfloat16)
    skv = K_ext.shape[1]
    V_aug = jnp.concatenate(
        [V_loc, jnp.ones((B, H_LOC, skv, 1), jnp.bfloat16)], axis=3
    )
    x_bf = x.astype(jnp.bfloat16)
    Wq_bf = (Wq * SCALE).astype(jnp.bfloat16)
    Wo_bf = Wo.astype(jnp.bfloat16)

    def body(x_ref, wq_ref, wo_ref, k_ref, v_ref, out_ref,
             comm_ref, send_sems, recv_sems):
        my_pos = lax.axis_index("i")
        partners = [lax.bitwise_xor(my_pos, k) for k in (1, 2, 3)]

        barrier_sem = pltpu.get_barrier_semaphore()
        for nbr in partners:
            pl.semaphore_signal(
                barrier_sem, inc=1,
                device_id=(nbr,), device_id_type=pl.DeviceIdType.MESH,
            )

        def partial_for_batch(b):
            xb = x_ref[b]
            q = lax.dot(xb, wq_ref[...],
                        preferred_element_type=jnp.float32
                        ).astype(jnp.bfloat16)
            o_cols = []
            for h in range(H_LOC):
                qh = q[:, h * DH:(h + 1) * DH]
                s = lax.dot_general(
                    qh, k_ref[b, h], (((1,), (1,)), ((), ())),
                    preferred_element_type=jnp.float32,
                )
                p = jnp.exp(s).astype(jnp.bfloat16)
                o = lax.dot(p, v_ref[b, h],
                            preferred_element_type=jnp.float32)
                r = 1.0 / o[:, DH:DH + 1]
                o_cols.append((o[:, :DH] * r).astype(jnp.bfloat16))
            attn = jnp.concatenate(o_cols, axis=1)
            return lax.dot(attn, wo_ref[...],
                           preferred_element_type=jnp.float32)

        def push(k, b):
            rdma = pltpu.make_async_remote_copy(
                src_ref=comm_ref.at[0, b],
                dst_ref=comm_ref.at[k, b],
                send_sem=send_sems.at[3 * b + k - 1],
                recv_sem=recv_sems.at[3 * b + k - 1],
                device_id=(partners[k - 1],),
                device_id_type=pl.DeviceIdType.MESH,
            )
            rdma.start()
            return rdma

        part0 = partial_for_batch(0)
        comm_ref[0, 0] = part0.astype(jnp.bfloat16)
        pl.semaphore_wait(barrier_sem, 3)
        sends0 = [push(k, 0) for k in (2, 1, 3)]

        part1 = partial_for_batch(1)
        comm_ref[0, 1] = part1.astype(jnp.bfloat16)
        sends1 = [push(k, 1) for k in (2, 1, 3)]

        for r in sends0:
            r.wait_recv()
        out_ref[0] = (part0
                      + comm_ref[1, 0].astype(jnp.float32)
                      + comm_ref[2, 0].astype(jnp.float32)
                      + comm_ref[3, 0].astype(jnp.float32))
        for r in sends1:
            r.wait_recv()
        out_ref[1] = (part1
                      + comm_ref[1, 1].astype(jnp.float32)
                      + comm_ref[2, 1].astype(jnp.float32)
                      + comm_ref[3, 1].astype(jnp.float32))

        for r in sends0 + sends1:
            r.wait_send()

    return pl.pallas_call(
        body,
        out_shape=jax.ShapeDtypeStruct((B, SQ, D), jnp.float32),
        in_specs=[pl.BlockSpec(memory_space=pltpu.VMEM)] * 5,
        out_specs=pl.BlockSpec(memory_space=pltpu.VMEM),
        scratch_shapes=[
            pltpu.VMEM((4, B, SQ, D), jnp.bfloat16),
            pltpu.SemaphoreType.DMA((6,)),
            pltpu.SemaphoreType.DMA((6,)),
        ],
        compiler_params=pltpu.CompilerParams(collective_id=0),
    )(x_bf, Wq_bf, Wo_bf, K_loc, V_aug)


# device time: 14295 ns/iter; 1.1228x vs baseline; 1.1228x over previous
import os

import jax
import jax.numpy as jnp
from jax import lax
from jax.experimental import pallas as pl
from jax.experimental.pallas import tpu as pltpu

N_DEV = 4
B, SQ, D = 2, 128, 512
SKV = 128
H_LOC = 8
DH = 64
SCALE = 0.125
DC = D

_NO_COMM = os.environ.get("SCB_NO_COMM") == "1"


def kernel(x, Wq, Wo, K_ext, V_ext):
    my = lax.axis_index("i")
    K2 = lax.dynamic_slice_in_dim(K_ext, my * H_LOC, H_LOC, axis=2
                                  ).reshape(B, SKV, H_LOC * DH)
    V2 = lax.dynamic_slice_in_dim(V_ext, my * H_LOC, H_LOC, axis=2
                                  ).reshape(B, SKV, H_LOC * DH)
    KV = jnp.concatenate([K2, V2], axis=-1).astype(jnp.bfloat16)

    def body(x_ref, wq_ref, wo_ref, kv_ref, out_ref,
             comm_ref, send_sems, recv_sems):
        my_pos = lax.axis_index("i")
        partners = [lax.bitwise_xor(my_pos, k) for k in (1, 2, 3)]

        barrier_sem = pltpu.get_barrier_semaphore()
        for nbr in partners:
            pl.semaphore_signal(
                barrier_sem, inc=1,
                device_id=(nbr,), device_id_type=pl.DeviceIdType.MESH,
            )

        wq = (wq_ref[...] * SCALE).astype(jnp.bfloat16)
        wo = wo_ref[...].astype(jnp.bfloat16)

        x_all = x_ref[...].reshape(B * SQ, D).astype(jnp.bfloat16)
        q_all = lax.dot(x_all, wq, preferred_element_type=jnp.float32
                        ).astype(jnp.bfloat16)

        def partial_for_batch(b):
            q = q_all[b * SQ:(b + 1) * SQ]
            kv = kv_ref[b]
            o_cols = []
            for h in range(H_LOC):
                qh = q[:, h * DH:(h + 1) * DH]
                kh = kv[:, h * DH:(h + 1) * DH]
                vh = kv[:, D + h * DH:D + (h + 1) * DH]
                s = lax.dot_general(
                    qh, kh, (((1,), (1,)), ((), ())),
                    preferred_element_type=jnp.float32,
                )
                p = jnp.exp(s)
                l = jnp.sum(p, axis=1, keepdims=True)
                o = lax.dot(p.astype(jnp.bfloat16), vh,
                            preferred_element_type=jnp.float32)
                o_cols.append((o * (1.0 / l)).astype(jnp.bfloat16))
            attn = jnp.concatenate(o_cols, axis=1)
            return lax.dot(attn, wo, preferred_element_type=jnp.float32)

        def push(k, b, c):
            rdma = pltpu.make_async_remote_copy(
                src_ref=comm_ref.at[0, b, :, pl.ds(c * DC, DC)],
                dst_ref=comm_ref.at[k, b, :, pl.ds(c * DC, DC)],
                send_sem=send_sems.at[3 * (2 * b + c) + k - 1],
                recv_sem=recv_sems.at[3 * (2 * b + c) + k - 1],
                device_id=(partners[k - 1],),
                device_id_type=pl.DeviceIdType.MESH,
            )
            rdma.start()
            return rdma

        sends = []

        def push_batch(b, part):
            comm_ref[0, b] = part.astype(jnp.bfloat16)
            sends.extend(push(k, b, 0) for k in (2, 1, 3))

        part0 = partial_for_batch(0)
        if not _NO_COMM:
            pl.semaphore_wait(barrier_sem, 3)
            push_batch(0, part0)

        part1 = partial_for_batch(1)
        if _NO_COMM:
            out_ref[0] = part0.astype(jnp.bfloat16)
            out_ref[1] = part1.astype(jnp.bfloat16)
            return
        push_batch(1, part1)

        for b, part in ((0, part0), (1, part1)):
            off = 3 * b
            sends[off + 1].wait_recv()
            sends[off + 2].wait_recv()
            acc = (part
                   + comm_ref[1, b].astype(jnp.float32)
                   + comm_ref[3, b].astype(jnp.float32))
            sends[off].wait_recv()
            out_ref[b] = (acc + comm_ref[2, b].astype(jnp.float32)
                          ).astype(jnp.bfloat16)

        for r in sends:
            r.wait_send()

    return pl.pallas_call(
        body,
        out_shape=jax.ShapeDtypeStruct((B, SQ, D), jnp.bfloat16),
        in_specs=[pl.BlockSpec(memory_space=pltpu.VMEM)] * 4,
        out_specs=pl.BlockSpec(memory_space=pltpu.VMEM),
        scratch_shapes=[
            pltpu.VMEM((4, B, SQ, D), jnp.bfloat16),
            pltpu.SemaphoreType.DMA((12,)),
            pltpu.SemaphoreType.DMA((12,)),
        ],
        compiler_params=pltpu.CompilerParams(collective_id=0),
    )(x, Wq, Wo, KV)
